# baseline (speedup 1.0000x reference)
import jax
import jax.numpy as jnp
import numpy as np
from jax.experimental import pallas as pl

_B = 1024
_IN = 1024
_HID = 512
_OUT = 256
_NSTEPS = 8
_T_LO, _T_HI = 1.0, 5.0


def _np_threefry2x32(k0, k1, x0, x1):
    R0 = (13, 15, 26, 6)
    R1 = (17, 29, 16, 24)
    ks0 = np.uint32(k0)
    ks1 = np.uint32(k1)
    ks2 = np.uint32(ks0 ^ ks1 ^ np.uint32(0x1BD11BDA))
    x0 = (x0 + ks0).astype(np.uint32)
    x1 = (x1 + ks1).astype(np.uint32)

    def rotl(x, r):
        return ((x << np.uint32(r)) | (x >> np.uint32(32 - r))).astype(np.uint32)

    ks = (ks0, ks1, ks2)
    rots = (R0, R1, R0, R1, R0)
    for i in range(5):
        for r in rots[i]:
            x0 = (x0 + x1).astype(np.uint32)
            x1 = rotl(x1, r) ^ x0
        x0 = (x0 + ks[(i + 1) % 3]).astype(np.uint32)
        x1 = (x1 + ks[(i + 2) % 3] + np.uint32(i + 1)).astype(np.uint32)
    return x0, x1


def _np_split3(kd):
    o0, o1 = _np_threefry2x32(
        kd[0], kd[1], np.zeros(3, np.uint32), np.arange(3, dtype=np.uint32)
    )
    return [(int(o0[i]), int(o1[i])) for i in range(3)]


def _key_chain():
    key = (0, 42)
    key, ka, kb = _np_split3(key)
    per_step = []
    for _ in range(_NSTEPS):
        key, k1, k2 = _np_split3(key)
        per_step.append((k1, k2))
    return ka, kb, per_step


_KA, _KB, _STEP_KEYS = _key_chain()


def _i32(v):
    v = int(v) & 0xFFFFFFFF
    return v - 0x100000000 if v >= 0x80000000 else v


def _rotl(x, r):
    left = jax.lax.shift_left(x, jax.lax.full_like(x, r))
    right = jax.lax.shift_right_logical(x, jax.lax.full_like(x, 32 - r))
    return jax.lax.bitwise_or(left, right)


def _flat_idx(r0, rows, cols):
    i = jax.lax.broadcasted_iota(jnp.int32, (rows, cols), 0)
    j = jax.lax.broadcasted_iota(jnp.int32, (rows, cols), 1)
    return (r0 + i) * np.int32(cols) + j


def _uniform_bits(key_pair, idx):
    k0, k1 = key_pair
    ks0 = _i32(k0)
    ks1 = _i32(k1)
    ks2 = _i32(k0 ^ k1 ^ 0x1BD11BDA)
    x0 = jax.lax.full_like(idx, ks0)
    x1 = idx + np.int32(ks1)
    ks = (ks0, ks1, ks2)
    rots = ((13, 15, 26, 6), (17, 29, 16, 24), (13, 15, 26, 6),
            (17, 29, 16, 24), (13, 15, 26, 6))
    for g in range(5):
        for r in rots[g]:
            x0 = x0 + x1
            x1 = _rotl(x1, r) ^ x0
        x0 = x0 + np.int32(ks[(g + 1) % 3])
        x1 = x1 + np.int32(_i32(ks[(g + 2) % 3] + g + 1))
    return x0 ^ x1


def _uniform(key_pair, idx):
    bits = _uniform_bits(key_pair, idx)
    mant = jax.lax.shift_right_logical(bits, jax.lax.full_like(bits, 9))
    fbits = jax.lax.bitwise_or(mant, jax.lax.full_like(mant, 0x3F800000))
    return jax.lax.bitcast_convert_type(fbits, jnp.float32) - np.float32(1.0)


def _init_state(key_pair, idx):
    bits = _uniform_bits(key_pair, idx)
    return (bits >= 0).astype(jnp.float32)


def _sample(p, key_pair, idx):
    u = _uniform(key_pair, idx)
    hard = (u < p).astype(jnp.float32)
    return p + (hard - p)


_ROWS = 512


def _body(x_ref, w0_ref, b0_ref, w1_ref, b1_ref, t_ref, s1_ref, s2_ref):
    r0 = pl.program_id(0) * _ROWS
    s0 = x_ref[...]
    w0 = w0_ref[...]
    b0 = b0_ref[...]
    w1 = w1_ref[...]
    b1 = b1_ref[...]
    tv = t_ref[...]

    idx_h = _flat_idx(r0, _ROWS, _HID)
    idx_o = _flat_idx(r0, _ROWS, _OUT)
    s1 = _init_state(_KA, idx_h)
    s2 = _init_state(_KB, idx_o)

    f32 = jnp.float32
    for step in range(_NSTEPS):
        t = tv[0, step]
        gap1 = (
            jax.lax.dot_general(s0, w0, (((1,), (1,)), ((), ())),
                                preferred_element_type=f32)
            + b0
            + jax.lax.dot_general(s2, w1, (((1,), (0,)), ((), ())),
                                  preferred_element_type=f32)
        )
        p1 = jax.nn.sigmoid(gap1 / t)
        s1 = _sample(p1, _STEP_KEYS[step][0], idx_h)
        gap2 = (
            jax.lax.dot_general(s1, w1, (((1,), (1,)), ((), ())),
                                preferred_element_type=f32)
            + b1
        )
        p2 = jax.nn.sigmoid(gap2 / t)
        s2 = _sample(p2, _STEP_KEYS[step][1], idx_o)

    s1_ref[...] = s1
    s2_ref[...] = s2


def kernel(x, W0, b0, W1, b1, steps):
    grad_t = (_T_HI - _T_LO) / steps
    ts = [
        jnp.asarray(grad_t * (steps - i - 1) + _T_LO, dtype=jnp.float32)
        for i in range(_NSTEPS)
    ]
    t_arr = jnp.stack(ts).reshape(1, _NSTEPS)

    nblk = _B // _ROWS
    s1, s2 = pl.pallas_call(
        _body,
        grid=(nblk,),
        in_specs=[
            pl.BlockSpec((_ROWS, _IN), lambda b: (b, 0)),
            pl.BlockSpec((_HID, _IN), lambda b: (0, 0)),
            pl.BlockSpec((1, _HID), lambda b: (0, 0)),
            pl.BlockSpec((_OUT, _HID), lambda b: (0, 0)),
            pl.BlockSpec((1, _OUT), lambda b: (0, 0)),
            pl.BlockSpec((1, _NSTEPS), lambda b: (0, 0)),
        ],
        out_specs=(
            pl.BlockSpec((_ROWS, _HID), lambda b: (b, 0)),
            pl.BlockSpec((_ROWS, _OUT), lambda b: (b, 0)),
        ),
        out_shape=(
            jax.ShapeDtypeStruct((_B, _HID), jnp.float32),
            jax.ShapeDtypeStruct((_B, _OUT), jnp.float32),
        ),
    )(x, W0, b0.reshape(1, _HID), W1, b1.reshape(1, _OUT), t_arr)
    return (x, s1, s2)


# speedup vs baseline: 1.3380x; 1.3380x over previous
import jax
import jax.numpy as jnp
import numpy as np
from jax.experimental import pallas as pl

_B = 1024
_IN = 1024
_HID = 512
_OUT = 256
_NSTEPS = 8
_T_LO, _T_HI = 1.0, 5.0


def _np_threefry2x32(k0, k1, x0, x1):
    R0 = (13, 15, 26, 6)
    R1 = (17, 29, 16, 24)
    ks0 = np.uint32(k0)
    ks1 = np.uint32(k1)
    ks2 = np.uint32(ks0 ^ ks1 ^ np.uint32(0x1BD11BDA))
    x0 = (x0 + ks0).astype(np.uint32)
    x1 = (x1 + ks1).astype(np.uint32)

    def rotl(x, r):
        return ((x << np.uint32(r)) | (x >> np.uint32(32 - r))).astype(np.uint32)

    ks = (ks0, ks1, ks2)
    rots = (R0, R1, R0, R1, R0)
    for i in range(5):
        for r in rots[i]:
            x0 = (x0 + x1).astype(np.uint32)
            x1 = rotl(x1, r) ^ x0
        x0 = (x0 + ks[(i + 1) % 3]).astype(np.uint32)
        x1 = (x1 + ks[(i + 2) % 3] + np.uint32(i + 1)).astype(np.uint32)
    return x0, x1


def _np_split3(kd):
    o0, o1 = _np_threefry2x32(
        kd[0], kd[1], np.zeros(3, np.uint32), np.arange(3, dtype=np.uint32)
    )
    return [(int(o0[i]), int(o1[i])) for i in range(3)]


def _key_chain():
    key = (0, 42)
    key, ka, kb = _np_split3(key)
    per_step = []
    for _ in range(_NSTEPS):
        key, k1, k2 = _np_split3(key)
        per_step.append((k1, k2))
    return ka, kb, per_step


_KA, _KB, _STEP_KEYS = _key_chain()


def _i32(v):
    v = int(v) & 0xFFFFFFFF
    return v - 0x100000000 if v >= 0x80000000 else v


def _rotl(x, r):
    left = jax.lax.shift_left(x, jax.lax.full_like(x, r))
    right = jax.lax.shift_right_logical(x, jax.lax.full_like(x, 32 - r))
    return jax.lax.bitwise_or(left, right)


def _flat_idx(r0, rows, cols):
    i = jax.lax.broadcasted_iota(jnp.int32, (rows, cols), 0)
    j = jax.lax.broadcasted_iota(jnp.int32, (rows, cols), 1)
    return (r0 + i) * np.int32(cols) + j


def _uniform_bits(key_pair, idx):
    k0, k1 = key_pair
    ks0 = _i32(k0)
    ks1 = _i32(k1)
    ks2 = _i32(k0 ^ k1 ^ 0x1BD11BDA)
    x0 = jax.lax.full_like(idx, ks0)
    x1 = idx + np.int32(ks1)
    ks = (ks0, ks1, ks2)
    rots = ((13, 15, 26, 6), (17, 29, 16, 24), (13, 15, 26, 6),
            (17, 29, 16, 24), (13, 15, 26, 6))
    for g in range(5):
        for r in rots[g]:
            x0 = x0 + x1
            x1 = _rotl(x1, r) ^ x0
        x0 = x0 + np.int32(ks[(g + 1) % 3])
        x1 = x1 + np.int32(_i32(ks[(g + 2) % 3] + g + 1))
    return x0 ^ x1


def _uniform(key_pair, idx):
    bits = _uniform_bits(key_pair, idx)
    mant = jax.lax.shift_right_logical(bits, jax.lax.full_like(bits, 9))
    fbits = jax.lax.bitwise_or(mant, jax.lax.full_like(mant, 0x3F800000))
    return jax.lax.bitcast_convert_type(fbits, jnp.float32) - np.float32(1.0)


def _init_state(key_pair, idx):
    bits = _uniform_bits(key_pair, idx)
    return (bits >= 0).astype(jnp.float32)


def _sample(p, key_pair, idx):
    u = _uniform(key_pair, idx)
    hard = (u < p).astype(jnp.float32)
    return p + (hard - p)


_ROWS = 256


def _body(x_ref, w0_ref, b0_ref, w1_ref, b1_ref, t_ref, s1_ref, s2_ref):
    r0 = pl.program_id(0) * _ROWS
    s0 = x_ref[...]
    w0 = w0_ref[...]
    b0 = b0_ref[...]
    w1 = w1_ref[...]
    b1 = b1_ref[...]
    tv = t_ref[...]

    idx_h = _flat_idx(r0, _ROWS, _HID)
    idx_o = _flat_idx(r0, _ROWS, _OUT)
    s1 = _init_state(_KA, idx_h)
    s2 = _init_state(_KB, idx_o)

    f32 = jnp.float32
    for step in range(_NSTEPS):
        t = tv[0, step]
        gap1 = (
            jax.lax.dot_general(s0, w0, (((1,), (1,)), ((), ())),
                                preferred_element_type=f32)
            + b0
            + jax.lax.dot_general(s2, w1, (((1,), (0,)), ((), ())),
                                  preferred_element_type=f32)
        )
        p1 = jax.nn.sigmoid(gap1 / t)
        s1 = _sample(p1, _STEP_KEYS[step][0], idx_h)
        gap2 = (
            jax.lax.dot_general(s1, w1, (((1,), (1,)), ((), ())),
                                preferred_element_type=f32)
            + b1
        )
        p2 = jax.nn.sigmoid(gap2 / t)
        s2 = _sample(p2, _STEP_KEYS[step][1], idx_o)

    s1_ref[...] = s1
    s2_ref[...] = s2


def kernel(x, W0, b0, W1, b1, steps):
    grad_t = (_T_HI - _T_LO) / steps
    ts = [
        jnp.asarray(grad_t * (steps - i - 1) + _T_LO, dtype=jnp.float32)
        for i in range(_NSTEPS)
    ]
    t_arr = jnp.stack(ts).reshape(1, _NSTEPS)

    nblk = _B // _ROWS
    s1, s2 = pl.pallas_call(
        _body,
        grid=(nblk,),
        in_specs=[
            pl.BlockSpec((_ROWS, _IN), lambda b: (b, 0)),
            pl.BlockSpec((_HID, _IN), lambda b: (0, 0)),
            pl.BlockSpec((1, _HID), lambda b: (0, 0)),
            pl.BlockSpec((_OUT, _HID), lambda b: (0, 0)),
            pl.BlockSpec((1, _OUT), lambda b: (0, 0)),
            pl.BlockSpec((1, _NSTEPS), lambda b: (0, 0)),
        ],
        out_specs=(
            pl.BlockSpec((_ROWS, _HID), lambda b: (b, 0)),
            pl.BlockSpec((_ROWS, _OUT), lambda b: (b, 0)),
        ),
        out_shape=(
            jax.ShapeDtypeStruct((_B, _HID), jnp.float32),
            jax.ShapeDtypeStruct((_B, _OUT), jnp.float32),
        ),
    )(x, W0, b0.reshape(1, _HID), W1, b1.reshape(1, _OUT), t_arr)
    return (x, s1, s2)
